# Initial kernel scaffold; baseline (speedup 1.0000x reference)
import jax
import jax.numpy as jnp
from jax.experimental import pallas as pl


def kernel(inputs, codebook):
    raise NotImplementedError("write your pallas kernel here")


# speedup vs baseline: 1.1075x; 1.1075x over previous
import jax
import jax.numpy as jnp
from jax import lax
from jax.experimental import pallas as pl

NUM_CODES = 1024
DIM = 64
PIX = 1024


def _vq_body(x_ref, cb_ref, zis_ref, zqs_ref):
    x = x_ref[...]
    cb = cb_ref[...]

    mm = lax.dot_general(cb, x, (((1,), (0,)), ((), ())),
                         precision=lax.Precision.DEFAULT)
    z2 = jnp.sum(x * x, axis=0)
    c2 = jnp.sum(cb * cb, axis=1)
    dist = (z2[None, :] + c2[:, None]) - 2.0 * mm

    m = jnp.min(dist, axis=0)
    ii = lax.broadcasted_iota(jnp.int32, (NUM_CODES, PIX), 0)
    idx = jnp.min(jnp.where(dist == m[None, :], ii, NUM_CODES), axis=0)

    zis_ref[...] = idx.reshape(8, 128)

    onehot = (ii == idx[None, :]).astype(jnp.float32)
    zq = lax.dot_general(cb, onehot, (((0,), (0,)), ((), ())),
                         precision=lax.Precision.DEFAULT)
    zqs_ref[...] = zq


def kernel(inputs, codebook):
    B = inputs.shape[0]
    x = inputs.reshape(B, DIM, PIX)
    zis, zqs = pl.pallas_call(
        _vq_body,
        grid=(B,),
        in_specs=[
            pl.BlockSpec((None, DIM, PIX), lambda b: (b, 0, 0)),
            pl.BlockSpec((NUM_CODES, DIM), lambda b: (0, 0)),
        ],
        out_specs=[
            pl.BlockSpec((None, 8, 128), lambda b: (b, 0, 0)),
            pl.BlockSpec((None, DIM, PIX), lambda b: (b, 0, 0)),
        ],
        out_shape=[
            jax.ShapeDtypeStruct((B, 8, 128), jnp.int32),
            jax.ShapeDtypeStruct((B, DIM, PIX), jnp.float32),
        ],
    )(x, codebook)
    return (zis.reshape(B, 32, 32), zqs.reshape(B, DIM, 32, 32))
